# Initial kernel scaffold; baseline (speedup 1.0000x reference)
import jax
import jax.numpy as jnp
from jax.experimental import pallas as pl


def kernel(x, batch, v):
    raise NotImplementedError("write your pallas kernel here")


# speedup vs baseline: 45.8559x; 45.8559x over previous
import functools

import jax
import jax.numpy as jnp
from jax import lax
from jax.experimental import pallas as pl
from jax.experimental.pallas import tpu as pltpu
from jax.experimental.pallas import tpu_sc as plsc

N = 262144
D = 3
T = 128
R = 128
RAD = 1.1
B = 64

P = 4096
C = 2048
NTG = 8
NBG = 4
HROWS = 16 * R
BBLK = 8


def _tc1_body(x_ref, b_ref, v_ref, a_ref, cnt_ref):
    i = pl.program_id(0)
    x = x_ref[...]
    v = v_ref[...]
    nh = (x[:, 0:1] * v[0:1, :]
          + x[:, 1:2] * v[1:2, :]
          + x[:, 2:3] * v[2:3, :])
    s = (nh + RAD) * (1.0 / (2.0 * RAD)) * R
    bin_ = jnp.clip(jnp.floor(s), 0.0, R - 1.0).astype(jnp.int32)
    bcol = b_ref[...]
    a_ref[...] = (bcol & (16 - 1)) * R + bin_
    bo = (bcol == lax.broadcasted_iota(jnp.int32, (1, B), 1)).astype(jnp.int32)
    c = jnp.sum(bo, axis=0, keepdims=True)

    @pl.when(i == 0)
    def _():
        cnt_ref[...] = jnp.zeros_like(cnt_ref)

    cnt_ref[...] += c


def _tc1(x, batch2, v):
    return pl.pallas_call(
        _tc1_body,
        grid=(N // P,),
        in_specs=[
            pl.BlockSpec((P, D), lambda i: (i, 0)),
            pl.BlockSpec((P, 1), lambda i: (i, 0)),
            pl.BlockSpec((D, T), lambda i: (0, 0)),
        ],
        out_specs=[
            pl.BlockSpec((P, T), lambda i: (i, 0)),
            pl.BlockSpec((1, B), lambda i: (0, 0)),
        ],
        out_shape=[
            jax.ShapeDtypeStruct((N + C, T), jnp.int32),
            jax.ShapeDtypeStruct((1, B), jnp.int32),
        ],
    )(x, batch2, v)


def _sc_hist_body(a_hbm, cnt_hbm, h_hbm, abuf, hist, cbuf):
    cid = lax.axis_index("c")
    sid = lax.axis_index("s")
    wid = sid * 2 + cid
    tg = wid // NBG
    bg = wid % NBG

    pltpu.sync_copy(cnt_hbm.at[0], cbuf)
    s1 = jnp.sum(cbuf[pl.ds(0, 16)])
    s2 = jnp.sum(cbuf[pl.ds(16, 16)])
    s3 = jnp.sum(cbuf[pl.ds(32, 16)])
    b1 = s1
    b2 = s1 + s2
    b3 = b2 + s3
    zero = jnp.int32(0)
    start = jnp.where(bg == 0, zero, jnp.where(bg == 1, b1, jnp.where(bg == 2, b2, b3)))
    end = jnp.where(bg == 0, b1, jnp.where(bg == 1, b2, jnp.where(bg == 2, b3, jnp.int32(N))))

    zeros16 = jnp.zeros((16,), jnp.float32)

    @pl.loop(0, HROWS)
    def _(i):
        hist[i] = zeros16

    ones16 = jnp.ones((16,), jnp.float32)
    iota16 = lax.broadcasted_iota(jnp.int32, (16,), 0)

    nch = (end - start + (C - 1)) // C

    def chunk_body(k, carry):
        p0 = start + k * C
        cnt = jnp.minimum(jnp.int32(C), end - p0)
        pltpu.sync_copy(a_hbm.at[pl.ds(p0, C), pl.ds(tg * 16, 16)], abuf)

        def pt(j, carry2):
            a = abuf[j]
            plsc.addupdate_scatter(hist, [a, iota16], ones16)
            return carry2

        lax.fori_loop(0, cnt, pt, 0)
        return carry

    lax.fori_loop(0, nch, chunk_body, 0)

    pltpu.sync_copy(hist, h_hbm.at[pl.ds(bg * HROWS, HROWS), tg, :])


def _tc2_body(h_ref, o_ref):
    def body(r, acc):
        acc = acc + h_ref[:, pl.ds(r, 1), :]
        o_ref[:, pl.ds(r, 1), :] = acc
        return acc

    lax.fori_loop(0, R, body, jnp.zeros((BBLK, 1, T), jnp.float32))


def _tc2(h3):
    return pl.pallas_call(
        _tc2_body,
        grid=(B // BBLK,),
        in_specs=[pl.BlockSpec((BBLK, R, T), lambda i: (i, 0, 0))],
        out_specs=pl.BlockSpec((BBLK, R, T), lambda i: (i, 0, 0)),
        out_shape=jax.ShapeDtypeStruct((B, R, T), jnp.float32),
    )(h3)


@functools.cache
def _sc_hist():
    mesh = plsc.VectorSubcoreMesh(core_axis_name="c", subcore_axis_name="s")
    return pl.kernel(
        _sc_hist_body,
        out_type=jax.ShapeDtypeStruct((NBG * HROWS, NTG, 16), jnp.float32),
        mesh=mesh,
        compiler_params=pltpu.CompilerParams(
            use_tc_tiling_on_sc=False, needs_layout_passes=False),
        scratch_types=[
            pltpu.VMEM((C, 16), jnp.int32),
            pltpu.VMEM((HROWS, 16), jnp.float32),
            pltpu.VMEM((B,), jnp.int32),
        ],
    )


def kernel(x, batch, v):
    batch2 = batch[:, None]
    a, counts = _tc1(x, batch2, v)
    h = _sc_hist()(a, counts)
    return _tc2(h.reshape(B, R, T))


# speedup vs baseline: 49.4454x; 1.0783x over previous
import functools

import jax
import jax.numpy as jnp
from jax import lax
from jax.experimental import pallas as pl
from jax.experimental.pallas import tpu as pltpu
from jax.experimental.pallas import tpu_sc as plsc

N = 262144
D = 3
T = 128
R = 128
RAD = 1.1
B = 64

P = 4096
C = 2048
NTG = 8
NBG = 4
HROWS = 16 * R
BBLK = 8


def _tc1_body(x_ref, b_ref, v_ref, a_ref, cnt_ref):
    i = pl.program_id(0)
    x = x_ref[...]
    v = v_ref[...]
    bcol = b_ref[...]
    k1 = R / (2.0 * RAD)
    x1 = jnp.concatenate([x, jnp.ones((P, 1), jnp.float32)], axis=1)
    v1 = jnp.concatenate([v * k1, jnp.full((1, T), R / 2.0)], axis=0)
    s = jnp.dot(x1, v1, preferred_element_type=jnp.float32)
    bloc = (bcol & (16 - 1)).astype(jnp.float32)
    base = jnp.dot(bloc, jnp.full((1, T), float(R)),
                   preferred_element_type=jnp.float32)
    a_ref[...] = (base + jnp.floor(jnp.clip(s, 0.0, R - 1.0))).astype(jnp.int32)
    bo = (bcol == lax.broadcasted_iota(jnp.int32, (1, B), 1)).astype(jnp.int32)
    c = jnp.sum(bo, axis=0, keepdims=True)

    @pl.when(i == 0)
    def _():
        cnt_ref[...] = jnp.zeros_like(cnt_ref)

    cnt_ref[...] += c


def _tc1(x, batch2, v):
    return pl.pallas_call(
        _tc1_body,
        grid=(N // P,),
        in_specs=[
            pl.BlockSpec((P, D), lambda i: (i, 0)),
            pl.BlockSpec((P, 1), lambda i: (i, 0)),
            pl.BlockSpec((D, T), lambda i: (0, 0)),
        ],
        out_specs=[
            pl.BlockSpec((P, T), lambda i: (i, 0)),
            pl.BlockSpec((1, B), lambda i: (0, 0)),
        ],
        out_shape=[
            jax.ShapeDtypeStruct((N + C, T), jnp.int32),
            jax.ShapeDtypeStruct((1, B), jnp.int32),
        ],
    )(x, batch2, v)


def _sc_hist_body(a_hbm, cnt_hbm, h_hbm, abuf, hist, cbuf):
    cid = lax.axis_index("c")
    sid = lax.axis_index("s")
    wid = sid * 2 + cid
    tg = wid // NBG
    bg = wid % NBG

    pltpu.sync_copy(cnt_hbm.at[0], cbuf)
    s1 = jnp.sum(cbuf[pl.ds(0, 16)])
    s2 = jnp.sum(cbuf[pl.ds(16, 16)])
    s3 = jnp.sum(cbuf[pl.ds(32, 16)])
    b1 = s1
    b2 = s1 + s2
    b3 = b2 + s3
    zero = jnp.int32(0)
    start = jnp.where(bg == 0, zero, jnp.where(bg == 1, b1, jnp.where(bg == 2, b2, b3)))
    end = jnp.where(bg == 0, b1, jnp.where(bg == 1, b2, jnp.where(bg == 2, b3, jnp.int32(N))))

    zeros16 = jnp.zeros((16,), jnp.float32)

    @pl.loop(0, HROWS + 1)
    def _(i):
        hist[i] = zeros16

    ones16 = jnp.ones((16,), jnp.float32)
    iota16 = lax.broadcasted_iota(jnp.int32, (16,), 0)
    trash16 = jnp.full((16,), HROWS, jnp.int32)

    nch = (end - start + (C - 1)) // C

    def chunk_body(k, carry):
        p0 = start + k * C
        cnt = jnp.minimum(jnp.int32(C), end - p0)
        pltpu.sync_copy(a_hbm.at[pl.ds(p0, C), pl.ds(tg * 16, 16)], abuf)

        def fill(j, carry2):
            abuf[j] = trash16
            return carry2

        lax.fori_loop(cnt, C, fill, 0)

        @pl.loop(0, C, step=16)
        def _(j):
            for u in range(16):
                a = abuf[j + u]
                plsc.addupdate_scatter(hist, [a, iota16], ones16)

        return carry

    lax.fori_loop(0, nch, chunk_body, 0)

    pltpu.sync_copy(hist.at[pl.ds(0, HROWS), :],
                    h_hbm.at[pl.ds(bg * HROWS, HROWS), tg, :])


def _tc2_body(h_ref, o_ref):
    def body(r, acc):
        acc = acc + h_ref[:, pl.ds(r, 1), :]
        o_ref[:, pl.ds(r, 1), :] = acc
        return acc

    lax.fori_loop(0, R, body, jnp.zeros((BBLK, 1, T), jnp.float32))


def _tc2(h3):
    return pl.pallas_call(
        _tc2_body,
        grid=(B // BBLK,),
        in_specs=[pl.BlockSpec((BBLK, R, T), lambda i: (i, 0, 0))],
        out_specs=pl.BlockSpec((BBLK, R, T), lambda i: (i, 0, 0)),
        out_shape=jax.ShapeDtypeStruct((B, R, T), jnp.float32),
    )(h3)


@functools.cache
def _sc_hist():
    mesh = plsc.VectorSubcoreMesh(core_axis_name="c", subcore_axis_name="s")
    return pl.kernel(
        _sc_hist_body,
        out_type=jax.ShapeDtypeStruct((NBG * HROWS, NTG, 16), jnp.float32),
        mesh=mesh,
        compiler_params=pltpu.CompilerParams(
            use_tc_tiling_on_sc=False, needs_layout_passes=False),
        scratch_types=[
            pltpu.VMEM((C, 16), jnp.int32),
            pltpu.VMEM((HROWS + 1, 16), jnp.float32),
            pltpu.VMEM((B,), jnp.int32),
        ],
    )


def kernel(x, batch, v):
    batch2 = batch[:, None]
    a, counts = _tc1(x, batch2, v)
    h = _sc_hist()(a, counts)
    return _tc2(h.reshape(B, R, T))


# speedup vs baseline: 87.4663x; 1.7689x over previous
import functools

import jax
import jax.numpy as jnp
from jax import lax
from jax.experimental import pallas as pl
from jax.experimental.pallas import tpu as pltpu
from jax.experimental.pallas import tpu_sc as plsc

N = 262144
D = 3
T = 128
R = 128
RAD = 1.1
B = 64

P = 4096
C = 2048
NTG = 8
NBG = 4
HROWS = 16 * R
BBLK = 8


def _tc1_body(x_ref, b_ref, v_ref, a_ref, cnt_ref):
    i = pl.program_id(0)
    x = x_ref[...]
    v = v_ref[...]
    bcol = b_ref[...]
    k1 = R / (2.0 * RAD)
    x1 = jnp.concatenate([x, jnp.ones((P, 1), jnp.float32)], axis=1)
    v1 = jnp.concatenate([v * k1, jnp.full((1, T), R / 2.0)], axis=0)
    s = jnp.dot(x1, v1, preferred_element_type=jnp.float32)
    bloc = (bcol & (16 - 1)).astype(jnp.float32)
    x2 = jnp.concatenate([bloc, jnp.ones((P, 1), jnp.float32)], axis=1)
    tlpat = (lax.broadcasted_iota(jnp.int32, (1, T), 1) & 15).astype(jnp.float32)
    v2 = jnp.concatenate([jnp.full((1, T), float(R * 16)), tlpat], axis=0)
    base = jnp.dot(x2, v2, preferred_element_type=jnp.float32)
    a_ref[...] = (base + jnp.floor(jnp.clip(s, 0.0, R - 1.0)) * 16.0
                  ).astype(jnp.int32)
    bo = (bcol == lax.broadcasted_iota(jnp.int32, (1, B), 1)).astype(jnp.int32)
    c = jnp.sum(bo, axis=0, keepdims=True)

    @pl.when(i == 0)
    def _():
        cnt_ref[...] = jnp.zeros_like(cnt_ref)

    cnt_ref[...] += c


def _tc1(x, batch2, v):
    return pl.pallas_call(
        _tc1_body,
        grid=(N // P,),
        in_specs=[
            pl.BlockSpec((P, D), lambda i: (i, 0)),
            pl.BlockSpec((P, 1), lambda i: (i, 0)),
            pl.BlockSpec((D, T), lambda i: (0, 0)),
        ],
        out_specs=[
            pl.BlockSpec((P, T), lambda i: (i, 0)),
            pl.BlockSpec((1, B), lambda i: (0, 0)),
        ],
        out_shape=[
            jax.ShapeDtypeStruct((N + C, T), jnp.int32),
            jax.ShapeDtypeStruct((1, B), jnp.int32),
        ],
    )(x, batch2, v)


def _sc_hist_body(a_hbm, cnt_hbm, h_hbm, abuf, hist, hist2, cbuf):
    cid = lax.axis_index("c")
    sid = lax.axis_index("s")
    wid = sid * 2 + cid
    tg = wid // NBG
    bg = wid % NBG

    pltpu.sync_copy(cnt_hbm.at[0], cbuf)
    s1 = jnp.sum(cbuf[pl.ds(0, 16)])
    s2 = jnp.sum(cbuf[pl.ds(16, 16)])
    s3 = jnp.sum(cbuf[pl.ds(32, 16)])
    b1 = s1
    b2 = s1 + s2
    b3 = b2 + s3
    zero = jnp.int32(0)
    start = jnp.where(bg == 0, zero, jnp.where(bg == 1, b1, jnp.where(bg == 2, b2, b3)))
    end = jnp.where(bg == 0, b1, jnp.where(bg == 1, b2, jnp.where(bg == 2, b3, jnp.int32(N))))

    zeros16 = jnp.zeros((16,), jnp.float32)

    @pl.loop(0, HROWS * 16 + 16, step=16)
    def _(i):
        hist[pl.ds(i, 16)] = zeros16

    ones16 = jnp.ones((16,), jnp.float32)
    iota16 = lax.broadcasted_iota(jnp.int32, (16,), 0)
    trash16 = jnp.full((16,), HROWS * 16, jnp.int32) + iota16

    nch = (end - start + (C - 1)) // C

    def chunk_body(k, carry):
        p0 = start + k * C
        cnt = jnp.minimum(jnp.int32(C), end - p0)
        pltpu.sync_copy(a_hbm.at[pl.ds(p0, C), pl.ds(tg * 16, 16)], abuf)

        def fill(j, carry2):
            abuf[j] = trash16
            return carry2

        lax.fori_loop(cnt, C, fill, 0)

        @pl.loop(0, C, step=16)
        def _(j):
            avs = [abuf[j + u] for u in range(16)]
            for a in avs:
                plsc.addupdate_scatter(hist, [a], ones16)

        return carry

    lax.fori_loop(0, nch, chunk_body, 0)

    @pl.loop(0, HROWS)
    def _(c):
        hist2[c] = hist[pl.ds(c * 16, 16)]

    pltpu.sync_copy(hist2, h_hbm.at[pl.ds(bg * HROWS, HROWS), tg, :])


def _tc2_body(h_ref, o_ref):
    def body(r, acc):
        acc = acc + h_ref[:, pl.ds(r, 1), :]
        o_ref[:, pl.ds(r, 1), :] = acc
        return acc

    lax.fori_loop(0, R, body, jnp.zeros((BBLK, 1, T), jnp.float32))


def _tc2(h3):
    return pl.pallas_call(
        _tc2_body,
        grid=(B // BBLK,),
        in_specs=[pl.BlockSpec((BBLK, R, T), lambda i: (i, 0, 0))],
        out_specs=pl.BlockSpec((BBLK, R, T), lambda i: (i, 0, 0)),
        out_shape=jax.ShapeDtypeStruct((B, R, T), jnp.float32),
    )(h3)


@functools.cache
def _sc_hist():
    mesh = plsc.VectorSubcoreMesh(core_axis_name="c", subcore_axis_name="s")
    return pl.kernel(
        _sc_hist_body,
        out_type=jax.ShapeDtypeStruct((NBG * HROWS, NTG, 16), jnp.float32),
        mesh=mesh,
        compiler_params=pltpu.CompilerParams(
            use_tc_tiling_on_sc=False, needs_layout_passes=False),
        scratch_types=[
            pltpu.VMEM((C, 16), jnp.int32),
            pltpu.VMEM((HROWS * 16 + 16,), jnp.float32),
            pltpu.VMEM((HROWS, 16), jnp.float32),
            pltpu.VMEM((B,), jnp.int32),
        ],
    )


def kernel(x, batch, v):
    batch2 = batch[:, None]
    a, counts = _tc1(x, batch2, v)
    h = _sc_hist()(a, counts)
    return _tc2(h.reshape(B, R, T))


# speedup vs baseline: 133.7205x; 1.5288x over previous
import functools

import jax
import jax.numpy as jnp
from jax import lax
from jax.experimental import pallas as pl
from jax.experimental.pallas import tpu as pltpu
from jax.experimental.pallas import tpu_sc as plsc

N = 262144
D = 3
T = 128
R = 128
RAD = 1.1
B = 64

P = 4096
C = 2048
NTG = 8
NBG = 4
HROWS = 16 * R
BBLK = 8


def _tc1_body(xt_ref, b_ref, v_ref, a_ref, cnt_ref):
    i = pl.program_id(0)
    xt = xt_ref[...]
    v = v_ref[...]
    br = b_ref[...]
    k1 = R / (2.0 * RAD)
    dn_t = (((0,), (0,)), ((), ()))
    x1t = jnp.concatenate([xt, jnp.ones((1, P), jnp.float32)], axis=0)
    v1 = jnp.concatenate([v * k1, jnp.full((1, T), R / 2.0)], axis=0)
    s = lax.dot_general(x1t, v1, dn_t,
                        preferred_element_type=jnp.float32)
    bloc = (br & (16 - 1)).astype(jnp.float32)
    x2t = jnp.concatenate([bloc, jnp.ones((1, P), jnp.float32)], axis=0)
    tlpat = (lax.broadcasted_iota(jnp.int32, (1, T), 1) & 15).astype(jnp.float32)
    v2 = jnp.concatenate([jnp.full((1, T), float(R * 16)), tlpat], axis=0)
    base = lax.dot_general(x2t, v2, dn_t,
                           preferred_element_type=jnp.float32)
    a_ref[...] = (base + jnp.floor(jnp.clip(s, 0.0, R - 1.0)) * 16.0
                  ).astype(jnp.int32)
    bo_t = (br == lax.broadcasted_iota(jnp.int32, (B, 1), 0)
            ).astype(jnp.float32)
    c = lax.dot_general(jnp.ones((1, P), jnp.float32), bo_t,
                        (((1,), (1,)), ((), ())),
                        preferred_element_type=jnp.float32)

    @pl.when(i == 0)
    def _():
        cnt_ref[...] = jnp.zeros_like(cnt_ref)

    cnt_ref[...] += c.astype(jnp.int32)


def _tc1(xt, br, v):
    return pl.pallas_call(
        _tc1_body,
        grid=(N // P,),
        in_specs=[
            pl.BlockSpec((D, P), lambda i: (0, i)),
            pl.BlockSpec((1, P), lambda i: (0, i)),
            pl.BlockSpec((D, T), lambda i: (0, 0)),
        ],
        out_specs=[
            pl.BlockSpec((P, T), lambda i: (i, 0)),
            pl.BlockSpec((1, B), lambda i: (0, 0)),
        ],
        out_shape=[
            jax.ShapeDtypeStruct((N + C, T), jnp.int32),
            jax.ShapeDtypeStruct((1, B), jnp.int32),
        ],
    )(xt, br, v)


def _sc_hist_body(a_hbm, cnt_hbm, h_hbm, abuf, hist, hist2, cbuf):
    cid = lax.axis_index("c")
    sid = lax.axis_index("s")
    wid = sid * 2 + cid
    tg = wid // NBG
    bg = wid % NBG

    pltpu.sync_copy(cnt_hbm.at[0], cbuf)
    s1 = jnp.sum(cbuf[pl.ds(0, 16)])
    s2 = jnp.sum(cbuf[pl.ds(16, 16)])
    s3 = jnp.sum(cbuf[pl.ds(32, 16)])
    b1 = s1
    b2 = s1 + s2
    b3 = b2 + s3
    zero = jnp.int32(0)
    start = jnp.where(bg == 0, zero, jnp.where(bg == 1, b1, jnp.where(bg == 2, b2, b3)))
    end = jnp.where(bg == 0, b1, jnp.where(bg == 1, b2, jnp.where(bg == 2, b3, jnp.int32(N))))

    zeros16 = jnp.zeros((16,), jnp.float32)

    @pl.loop(0, HROWS * 16 + 16, step=16)
    def _(i):
        hist[pl.ds(i, 16)] = zeros16

    ones16 = jnp.ones((16,), jnp.float32)
    iota16 = lax.broadcasted_iota(jnp.int32, (16,), 0)
    trash16 = jnp.full((16,), HROWS * 16, jnp.int32) + iota16

    nch = (end - start + (C - 1)) // C

    def chunk_body(k, carry):
        p0 = start + k * C
        cnt = jnp.minimum(jnp.int32(C), end - p0)
        pltpu.sync_copy(a_hbm.at[pl.ds(p0, C), pl.ds(tg * 16, 16)], abuf)

        def fill(j, carry2):
            abuf[j] = trash16
            return carry2

        lax.fori_loop(cnt, C, fill, 0)

        @pl.loop(0, C, step=16)
        def _(j):
            avs = [abuf[j + u] for u in range(16)]
            for a in avs:
                plsc.addupdate_scatter(hist, [a], ones16)

        return carry

    lax.fori_loop(0, nch, chunk_body, 0)

    @pl.loop(0, HROWS)
    def _(c):
        hist2[c] = hist[pl.ds(c * 16, 16)]

    pltpu.sync_copy(hist2, h_hbm.at[pl.ds(bg * HROWS, HROWS), tg, :])


def _tc2_body(h_ref, o_ref):
    def body(r, acc):
        acc = acc + h_ref[:, pl.ds(r, 1), :]
        o_ref[:, pl.ds(r, 1), :] = acc
        return acc

    lax.fori_loop(0, R, body, jnp.zeros((BBLK, 1, T), jnp.float32))


def _tc2(h3):
    return pl.pallas_call(
        _tc2_body,
        grid=(B // BBLK,),
        in_specs=[pl.BlockSpec((BBLK, R, T), lambda i: (i, 0, 0))],
        out_specs=pl.BlockSpec((BBLK, R, T), lambda i: (i, 0, 0)),
        out_shape=jax.ShapeDtypeStruct((B, R, T), jnp.float32),
    )(h3)


@functools.cache
def _sc_hist():
    mesh = plsc.VectorSubcoreMesh(core_axis_name="c", subcore_axis_name="s")
    return pl.kernel(
        _sc_hist_body,
        out_type=jax.ShapeDtypeStruct((NBG * HROWS, NTG, 16), jnp.float32),
        mesh=mesh,
        compiler_params=pltpu.CompilerParams(
            use_tc_tiling_on_sc=False, needs_layout_passes=False),
        scratch_types=[
            pltpu.VMEM((C, 16), jnp.int32),
            pltpu.VMEM((HROWS * 16 + 16,), jnp.float32),
            pltpu.VMEM((HROWS, 16), jnp.float32),
            pltpu.VMEM((B,), jnp.int32),
        ],
    )


def kernel(x, batch, v):
    a, counts = _tc1(x.T, batch[None, :], v)
    h = _sc_hist()(a, counts)
    return _tc2(h.reshape(B, R, T))


# speedup vs baseline: 180.9794x; 1.3534x over previous
import functools

import jax
import jax.numpy as jnp
from jax import lax
from jax.experimental import pallas as pl
from jax.experimental.pallas import tpu as pltpu
from jax.experimental.pallas import tpu_sc as plsc

N = 262144
D = 3
T = 128
R = 128
RAD = 1.1
B = 64

P = 4096
C = 1536
NTG = 8
NBG = 4
HROWS = 16 * R
BBLK = 8


def _tc1_body(xt_ref, b_ref, v_ref, a_ref, cnt_ref):
    i = pl.program_id(0)
    xt = xt_ref[...]
    v = v_ref[...]
    br = b_ref[...]
    k1 = R / (2.0 * RAD)
    dn_t = (((0,), (0,)), ((), ()))
    x1t = jnp.concatenate([xt, jnp.ones((1, P), jnp.float32)], axis=0)
    v1 = jnp.concatenate([v * k1, jnp.full((1, T), R / 2.0)], axis=0)
    s = lax.dot_general(x1t, v1, dn_t,
                        preferred_element_type=jnp.float32)
    bloc = (br & (16 - 1)).astype(jnp.float32)
    x2t = jnp.concatenate([bloc, jnp.ones((1, P), jnp.float32)], axis=0)
    tlpat = (lax.broadcasted_iota(jnp.int32, (1, T), 1) & 15).astype(jnp.float32)
    v2 = jnp.concatenate([jnp.full((1, T), float(R * 16)), tlpat], axis=0)
    base = lax.dot_general(x2t, v2, dn_t,
                           preferred_element_type=jnp.float32)
    a_ref[...] = (base + jnp.floor(jnp.clip(s, 0.0, R - 1.0)) * 16.0
                  ).astype(jnp.int32)
    bo_t = (br == lax.broadcasted_iota(jnp.int32, (B, 1), 0)
            ).astype(jnp.float32)
    c = lax.dot_general(jnp.ones((1, P), jnp.float32), bo_t,
                        (((1,), (1,)), ((), ())),
                        preferred_element_type=jnp.float32)

    @pl.when(i == 0)
    def _():
        cnt_ref[...] = jnp.zeros_like(cnt_ref)

    cnt_ref[...] += c.astype(jnp.int32)


def _tc1(xt, br, v):
    return pl.pallas_call(
        _tc1_body,
        grid=(N // P,),
        in_specs=[
            pl.BlockSpec((D, P), lambda i: (0, i)),
            pl.BlockSpec((1, P), lambda i: (0, i)),
            pl.BlockSpec((D, T), lambda i: (0, 0)),
        ],
        out_specs=[
            pl.BlockSpec((P, T), lambda i: (i, 0)),
            pl.BlockSpec((1, B), lambda i: (0, 0)),
        ],
        out_shape=[
            jax.ShapeDtypeStruct((N + C, T), jnp.int32),
            jax.ShapeDtypeStruct((1, B), jnp.int32),
        ],
    )(xt, br, v)


def _sc_hist_body(a_hbm, cnt_hbm, h_hbm, abufa, abufb, hist, hist2, cbuf,
                  sema, semb):
    cid = lax.axis_index("c")
    sid = lax.axis_index("s")
    wid = sid * 2 + cid
    tg = wid // NBG
    bg = wid % NBG

    pltpu.sync_copy(cnt_hbm.at[0], cbuf)
    s1 = jnp.sum(cbuf[pl.ds(0, 16)])
    s2 = jnp.sum(cbuf[pl.ds(16, 16)])
    s3 = jnp.sum(cbuf[pl.ds(32, 16)])
    b1 = s1
    b2 = s1 + s2
    b3 = b2 + s3
    zero = jnp.int32(0)
    start = jnp.where(bg == 0, zero, jnp.where(bg == 1, b1, jnp.where(bg == 2, b2, b3)))
    end = jnp.where(bg == 0, b1, jnp.where(bg == 1, b2, jnp.where(bg == 2, b3, jnp.int32(N))))

    ones16 = jnp.ones((16,), jnp.float32)
    iota16 = lax.broadcasted_iota(jnp.int32, (16,), 0)
    trash16 = jnp.full((16,), HROWS * 16, jnp.int32) + iota16

    nch = (end - start + (C - 1)) // C
    npair = jnp.maximum(jnp.int32(1), (nch + 1) // 2)

    def issue(k, buf, sem):
        p0 = jnp.minimum(start + k * C, jnp.int32(N))
        pltpu.make_async_copy(
            a_hbm.at[pl.ds(p0, C), pl.ds(tg * 16, 16)], buf, sem).start()

    def process(k, buf, sem):
        pltpu.make_async_copy(
            a_hbm.at[pl.ds(0, C), pl.ds(0, 16)], buf, sem).wait()
        cnt = jnp.clip(end - (start + k * C), 0, C)

        def fill(j, carry2):
            buf[j] = trash16
            return carry2

        lax.fori_loop(cnt, C, fill, 0)

        @pl.loop(0, C, step=16)
        def _(j):
            avs = [buf[j + u] for u in range(16)]
            for a in avs:
                plsc.addupdate_scatter(hist, [a], ones16)

    issue(jnp.int32(0), abufa, sema)
    issue(jnp.int32(1), abufb, semb)

    zeros16 = jnp.zeros((16,), jnp.float32)

    @pl.loop(0, HROWS * 16 + 16, step=16)
    def _(i):
        hist[pl.ds(i, 16)] = zeros16

    def pair_body(m, carry):
        more = m + 1 < npair
        process(2 * m, abufa, sema)

        @pl.when(more)
        def _():
            issue(2 * m + 2, abufa, sema)

        process(2 * m + 1, abufb, semb)

        @pl.when(more)
        def _():
            issue(2 * m + 3, abufb, semb)

        return carry

    lax.fori_loop(0, npair, pair_body, 0)

    @pl.loop(0, HROWS)
    def _(c):
        hist2[c] = hist[pl.ds(c * 16, 16)]

    pltpu.sync_copy(hist2, h_hbm.at[pl.ds(bg * HROWS, HROWS), tg, :])


def _tc2_body(h_ref, o_ref):
    def body(r, acc):
        acc = acc + h_ref[:, pl.ds(r, 1), :]
        o_ref[:, pl.ds(r, 1), :] = acc
        return acc

    lax.fori_loop(0, R, body, jnp.zeros((BBLK, 1, T), jnp.float32))


def _tc2(h3):
    return pl.pallas_call(
        _tc2_body,
        grid=(B // BBLK,),
        in_specs=[pl.BlockSpec((BBLK, R, T), lambda i: (i, 0, 0))],
        out_specs=pl.BlockSpec((BBLK, R, T), lambda i: (i, 0, 0)),
        out_shape=jax.ShapeDtypeStruct((B, R, T), jnp.float32),
    )(h3)


@functools.cache
def _sc_hist():
    mesh = plsc.VectorSubcoreMesh(core_axis_name="c", subcore_axis_name="s")
    return pl.kernel(
        _sc_hist_body,
        out_type=jax.ShapeDtypeStruct((NBG * HROWS, NTG, 16), jnp.float32),
        mesh=mesh,
        compiler_params=pltpu.CompilerParams(
            use_tc_tiling_on_sc=False, needs_layout_passes=False),
        scratch_types=[
            pltpu.VMEM((C, 16), jnp.int32),
            pltpu.VMEM((C, 16), jnp.int32),
            pltpu.VMEM((HROWS * 16 + 16,), jnp.float32),
            pltpu.VMEM((HROWS, 16), jnp.float32),
            pltpu.VMEM((B,), jnp.int32),
            pltpu.SemaphoreType.DMA,
            pltpu.SemaphoreType.DMA,
        ],
    )


def kernel(x, batch, v):
    a, counts = _tc1(x.T, batch[None, :], v)
    h = _sc_hist()(a, counts)
    return _tc2(h.reshape(B, R, T))


# speedup vs baseline: 189.7991x; 1.0487x over previous
import functools

import jax
import jax.numpy as jnp
from jax import lax
from jax.experimental import pallas as pl
from jax.experimental.pallas import tpu as pltpu
from jax.experimental.pallas import tpu_sc as plsc

N = 262144
D = 3
T = 128
R = 128
RAD = 1.1
B = 64

P = 4096
C = 1536
NTG = 8
NBG = 4
HROWS = 16 * R
BBLK = 8


def _tc1_body(xt_ref, b_ref, v_ref, a_ref, cnt_ref):
    i = pl.program_id(0)
    xt = xt_ref[...]
    v = v_ref[...]
    br = b_ref[...]
    k1 = R / (2.0 * RAD)
    dn_t = (((0,), (0,)), ((), ()))
    x1t = jnp.concatenate([xt, jnp.ones((1, P), jnp.float32)], axis=0)
    v1 = jnp.concatenate([v * k1, jnp.full((1, T), R / 2.0)], axis=0)
    s = lax.dot_general(x1t, v1, dn_t,
                        preferred_element_type=jnp.float32)
    bloc = (br & (16 - 1)).astype(jnp.float32)
    x2t = jnp.concatenate([bloc, jnp.ones((1, P), jnp.float32)], axis=0)
    tlpat = (lax.broadcasted_iota(jnp.int32, (1, T), 1) & 15).astype(jnp.float32)
    v2 = jnp.concatenate([jnp.full((1, T), float(R * 16)), tlpat], axis=0)
    base = lax.dot_general(x2t, v2, dn_t,
                           preferred_element_type=jnp.float32)
    a_ref[...] = (base + jnp.floor(jnp.clip(s, 0.0, R - 1.0)) * 16.0
                  ).astype(jnp.int32)
    thr = (lax.broadcasted_iota(jnp.int32, (16, 1), 0) + 1) * 16
    bo_t = (br < thr).astype(jnp.float32)
    c = lax.dot_general(jnp.ones((1, P), jnp.float32), bo_t,
                        (((1,), (1,)), ((), ())),
                        preferred_element_type=jnp.float32)

    @pl.when(i == 0)
    def _():
        cnt_ref[...] = jnp.zeros_like(cnt_ref)

    cnt_ref[...] += c.astype(jnp.int32)


def _tc1(xt, br, v):
    return pl.pallas_call(
        _tc1_body,
        grid=(N // P,),
        in_specs=[
            pl.BlockSpec((D, P), lambda i: (0, i)),
            pl.BlockSpec((1, P), lambda i: (0, i)),
            pl.BlockSpec((D, T), lambda i: (0, 0)),
        ],
        out_specs=[
            pl.BlockSpec((P, T), lambda i: (i, 0)),
            pl.BlockSpec((1, 16), lambda i: (0, 0)),
        ],
        out_shape=[
            jax.ShapeDtypeStruct((N + C, T), jnp.int32),
            jax.ShapeDtypeStruct((1, 16), jnp.int32),
        ],
    )(xt, br, v)


def _sc_hist_body(a_hbm, cnt_hbm, h_hbm, abufa, abufb, hist, hist2, cbuf,
                  sema, semb):
    cid = lax.axis_index("c")
    sid = lax.axis_index("s")
    wid = sid * 2 + cid
    tg = wid // NBG
    bg = wid % NBG

    pltpu.sync_copy(cnt_hbm.at[0], cbuf)
    iota16 = lax.broadcasted_iota(jnp.int32, (16,), 0)
    cb = cbuf[...]
    zero = jnp.int32(0)
    b1 = jnp.sum(jnp.where(iota16 == 0, cb, zero))
    b2 = jnp.sum(jnp.where(iota16 == 1, cb, zero))
    b3 = jnp.sum(jnp.where(iota16 == 2, cb, zero))
    start = jnp.where(bg == 0, zero, jnp.where(bg == 1, b1, jnp.where(bg == 2, b2, b3)))
    end = jnp.where(bg == 0, b1, jnp.where(bg == 1, b2, jnp.where(bg == 2, b3, jnp.int32(N))))

    ones16 = jnp.ones((16,), jnp.float32)
    trash16 = jnp.full((16,), HROWS * 16, jnp.int32) + iota16

    nch = (end - start + (C - 1)) // C
    npair = jnp.maximum(jnp.int32(1), (nch + 1) // 2)

    def issue(k, buf, sem):
        p0 = jnp.minimum(start + k * C, jnp.int32(N))
        pltpu.make_async_copy(
            a_hbm.at[pl.ds(p0, C), pl.ds(tg * 16, 16)], buf, sem).start()

    def process(k, buf, sem):
        pltpu.make_async_copy(
            a_hbm.at[pl.ds(0, C), pl.ds(0, 16)], buf, sem).wait()
        cnt = jnp.clip(end - (start + k * C), 0, C)

        def fill(j, carry2):
            buf[j] = trash16
            return carry2

        lax.fori_loop(cnt, C, fill, 0)

        @pl.loop(0, C, step=16)
        def _(j):
            avs = [buf[j + u] for u in range(16)]
            for a in avs:
                plsc.addupdate_scatter(hist, [a], ones16)

    issue(jnp.int32(0), abufa, sema)
    issue(jnp.int32(1), abufb, semb)

    zeros16 = jnp.zeros((16,), jnp.float32)

    @pl.loop(0, HROWS * 16 + 16, step=16)
    def _(i):
        hist[pl.ds(i, 16)] = zeros16

    def pair_body(m, carry):
        more = m + 1 < npair
        process(2 * m, abufa, sema)

        @pl.when(more)
        def _():
            issue(2 * m + 2, abufa, sema)

        process(2 * m + 1, abufb, semb)

        @pl.when(more)
        def _():
            issue(2 * m + 3, abufb, semb)

        return carry

    lax.fori_loop(0, npair, pair_body, 0)

    for bl in range(16):
        def csum(r, acc):
            acc = acc + hist[pl.ds((bl * R + r) * 16, 16)]
            hist2[bl * R + r] = acc
            return acc

        lax.fori_loop(0, R, csum, jnp.zeros((16,), jnp.float32))

    pltpu.sync_copy(hist2, h_hbm.at[pl.ds(bg * HROWS, HROWS), tg, :])


@functools.cache
def _sc_hist():
    mesh = plsc.VectorSubcoreMesh(core_axis_name="c", subcore_axis_name="s")
    return pl.kernel(
        _sc_hist_body,
        out_type=jax.ShapeDtypeStruct((NBG * HROWS, NTG, 16), jnp.float32),
        mesh=mesh,
        compiler_params=pltpu.CompilerParams(
            use_tc_tiling_on_sc=False, needs_layout_passes=False),
        scratch_types=[
            pltpu.VMEM((C, 16), jnp.int32),
            pltpu.VMEM((C, 16), jnp.int32),
            pltpu.VMEM((HROWS * 16 + 16,), jnp.float32),
            pltpu.VMEM((HROWS, 16), jnp.float32),
            pltpu.VMEM((16,), jnp.int32),
            pltpu.SemaphoreType.DMA,
            pltpu.SemaphoreType.DMA,
        ],
    )


def kernel(x, batch, v):
    a, counts = _tc1(x.T, batch[None, :], v)
    h = _sc_hist()(a, counts)
    return h.reshape(B, R, T)


# speedup vs baseline: 199.9459x; 1.0535x over previous
import functools

import jax
import jax.numpy as jnp
from jax import lax
from jax.experimental import pallas as pl
from jax.experimental.pallas import tpu as pltpu
from jax.experimental.pallas import tpu_sc as plsc

N = 262144
D = 3
T = 128
R = 128
RAD = 1.1
B = 64

P = 4096
C = 1536
NTG = 8
NBG = 4
HROWS = 16 * R
BBLK = 8


def _tc1_body(xt_ref, b_ref, v_ref, a_ref, cnt_ref):
    i = pl.program_id(0)
    xt = xt_ref[...]
    v = v_ref[...]
    br = b_ref[...]
    k1 = R / (2.0 * RAD)
    dn_t = (((0,), (0,)), ((), ()))
    bloc = (br & (16 - 1)).astype(jnp.float32)
    lhs = jnp.concatenate([xt, jnp.ones((1, P), jnp.float32), bloc],
                          axis=0)
    tlpat = (lax.broadcasted_iota(jnp.int32, (1, T), 1) & 15).astype(jnp.float32)
    vs = jnp.concatenate([v * k1, jnp.full((1, T), R / 2.0),
                          jnp.zeros((1, T), jnp.float32)], axis=0)
    vb = jnp.concatenate([jnp.zeros((3, T), jnp.float32), tlpat,
                          jnp.full((1, T), float(R * 16))], axis=0)
    sb = lax.dot_general(lhs, jnp.concatenate([vs, vb], axis=1), dn_t,
                         preferred_element_type=jnp.float32)
    s = sb[:, :T]
    base = sb[:, T:]
    a_ref[...] = (base + jnp.floor(jnp.clip(s, 0.0, R - 1.0)) * 16.0
                  ).astype(jnp.int32)
    thr = (lax.broadcasted_iota(jnp.int32, (16, 1), 0) + 1) * 16
    bo_t = (br < thr).astype(jnp.float32)
    c = lax.dot_general(jnp.ones((1, P), jnp.float32), bo_t,
                        (((1,), (1,)), ((), ())),
                        preferred_element_type=jnp.float32)

    @pl.when(i == 0)
    def _():
        cnt_ref[...] = jnp.zeros_like(cnt_ref)

    cnt_ref[...] += c.astype(jnp.int32)


def _tc1(xt, br, v):
    return pl.pallas_call(
        _tc1_body,
        grid=(N // P,),
        in_specs=[
            pl.BlockSpec((D, P), lambda i: (0, i)),
            pl.BlockSpec((1, P), lambda i: (0, i)),
            pl.BlockSpec((D, T), lambda i: (0, 0)),
        ],
        out_specs=[
            pl.BlockSpec((P, T), lambda i: (i, 0)),
            pl.BlockSpec((1, 16), lambda i: (0, 0)),
        ],
        out_shape=[
            jax.ShapeDtypeStruct((N + C, T), jnp.int32),
            jax.ShapeDtypeStruct((1, 16), jnp.int32),
        ],
    )(xt, br, v)


def _sc_hist_body(a_hbm, cnt_hbm, h_hbm, abufa, abufb, hist, hist2, cbuf,
                  sema, semb):
    cid = lax.axis_index("c")
    sid = lax.axis_index("s")
    wid = sid * 2 + cid
    tg = wid // NBG
    bg = wid % NBG

    pltpu.sync_copy(cnt_hbm.at[0], cbuf)
    iota16 = lax.broadcasted_iota(jnp.int32, (16,), 0)
    cb = cbuf[...]
    zero = jnp.int32(0)
    b1 = jnp.sum(jnp.where(iota16 == 0, cb, zero))
    b2 = jnp.sum(jnp.where(iota16 == 1, cb, zero))
    b3 = jnp.sum(jnp.where(iota16 == 2, cb, zero))
    start = jnp.where(bg == 0, zero, jnp.where(bg == 1, b1, jnp.where(bg == 2, b2, b3)))
    end = jnp.where(bg == 0, b1, jnp.where(bg == 1, b2, jnp.where(bg == 2, b3, jnp.int32(N))))

    ones16 = jnp.ones((16,), jnp.float32)
    trash16 = jnp.full((16,), HROWS * 16, jnp.int32) + iota16

    nch = (end - start + (C - 1)) // C
    npair = jnp.maximum(jnp.int32(1), (nch + 1) // 2)

    def issue(k, buf, sem):
        p0 = jnp.minimum(start + k * C, jnp.int32(N))
        pltpu.make_async_copy(
            a_hbm.at[pl.ds(p0, C), pl.ds(tg * 16, 16)], buf, sem).start()

    def process(k, buf, sem):
        pltpu.make_async_copy(
            a_hbm.at[pl.ds(0, C), pl.ds(0, 16)], buf, sem).wait()
        cnt = jnp.clip(end - (start + k * C), 0, C)

        def fill(j, carry2):
            buf[j] = trash16
            return carry2

        lax.fori_loop(cnt, C, fill, 0)

        @pl.loop(0, C, step=16)
        def _(j):
            avs = [buf[j + u] for u in range(16)]
            for a in avs:
                plsc.addupdate_scatter(hist, [a], ones16)

    issue(jnp.int32(0), abufa, sema)
    issue(jnp.int32(1), abufb, semb)

    zeros16 = jnp.zeros((16,), jnp.float32)

    @pl.loop(0, HROWS * 16 + 16, step=16)
    def _(i):
        hist[pl.ds(i, 16)] = zeros16

    def pair_body(m, carry):
        more = m + 1 < npair
        process(2 * m, abufa, sema)

        @pl.when(more)
        def _():
            issue(2 * m + 2, abufa, sema)

        process(2 * m + 1, abufb, semb)

        @pl.when(more)
        def _():
            issue(2 * m + 3, abufb, semb)

        return carry

    lax.fori_loop(0, npair, pair_body, 0)

    for bl in range(16):
        def csum(r, acc):
            acc = acc + hist[pl.ds((bl * R + r) * 16, 16)]
            hist2[bl * R + r] = acc
            return acc

        lax.fori_loop(0, R, csum, jnp.zeros((16,), jnp.float32))

    pltpu.sync_copy(hist2, h_hbm.at[pl.ds(bg * HROWS, HROWS), tg, :])


@functools.cache
def _sc_hist():
    mesh = plsc.VectorSubcoreMesh(core_axis_name="c", subcore_axis_name="s")
    return pl.kernel(
        _sc_hist_body,
        out_type=jax.ShapeDtypeStruct((NBG * HROWS, NTG, 16), jnp.float32),
        mesh=mesh,
        compiler_params=pltpu.CompilerParams(
            use_tc_tiling_on_sc=False, needs_layout_passes=False),
        scratch_types=[
            pltpu.VMEM((C, 16), jnp.int32),
            pltpu.VMEM((C, 16), jnp.int32),
            pltpu.VMEM((HROWS * 16 + 16,), jnp.float32),
            pltpu.VMEM((HROWS, 16), jnp.float32),
            pltpu.VMEM((16,), jnp.int32),
            pltpu.SemaphoreType.DMA,
            pltpu.SemaphoreType.DMA,
        ],
    )


def kernel(x, batch, v):
    a, counts = _tc1(x.T, batch[None, :], v)
    h = _sc_hist()(a, counts)
    return h.reshape(B, R, T)


# speedup vs baseline: 218.8632x; 1.0946x over previous
import functools

import jax
import jax.numpy as jnp
from jax import lax
from jax.experimental import pallas as pl
from jax.experimental.pallas import tpu as pltpu
from jax.experimental.pallas import tpu_sc as plsc

N = 262144
D = 3
T = 128
R = 128
RAD = 1.1
B = 64

P = 8192
C = 1536
NTG = 8
NBG = 4
HROWS = 16 * R
BBLK = 8


def _tc1_body(xt_ref, b_ref, v_ref, a_ref, cnt_ref):
    i = pl.program_id(0)
    xt = xt_ref[...]
    v = v_ref[...]
    br = b_ref[...]
    k1 = R / (2.0 * RAD)
    dn_t = (((0,), (0,)), ((), ()))
    bloc = (br & (16 - 1)).astype(jnp.float32)
    lhs = jnp.concatenate([xt, jnp.ones((1, P), jnp.float32), bloc],
                          axis=0)
    tlpat = (lax.broadcasted_iota(jnp.int32, (1, T), 1) & 15).astype(jnp.float32)
    vs = jnp.concatenate([v * k1, jnp.full((1, T), R / 2.0),
                          jnp.zeros((1, T), jnp.float32)], axis=0)
    vb = jnp.concatenate([jnp.zeros((3, T), jnp.float32), tlpat,
                          jnp.full((1, T), float(R * 16))], axis=0)
    sb = lax.dot_general(lhs, jnp.concatenate([vs, vb], axis=1), dn_t,
                         preferred_element_type=jnp.float32)
    s = sb[:, :T]
    base = sb[:, T:]
    a_ref[...] = (base + jnp.floor(jnp.clip(s, 0.0, R - 1.0)) * 16.0
                  ).astype(jnp.int32)
    thr = (lax.broadcasted_iota(jnp.int32, (16, 1), 0) + 1) * 16
    bo_t = (br < thr).astype(jnp.float32)
    c = lax.dot_general(jnp.ones((1, P), jnp.float32), bo_t,
                        (((1,), (1,)), ((), ())),
                        preferred_element_type=jnp.float32)

    @pl.when(i == 0)
    def _():
        cnt_ref[...] = jnp.zeros_like(cnt_ref)

    cnt_ref[...] += c.astype(jnp.int32)


def _tc1(xt, br, v):
    return pl.pallas_call(
        _tc1_body,
        grid=(N // P,),
        in_specs=[
            pl.BlockSpec((D, P), lambda i: (0, i)),
            pl.BlockSpec((1, P), lambda i: (0, i)),
            pl.BlockSpec((D, T), lambda i: (0, 0)),
        ],
        out_specs=[
            pl.BlockSpec((P, T), lambda i: (i, 0)),
            pl.BlockSpec((1, 16), lambda i: (0, 0)),
        ],
        out_shape=[
            jax.ShapeDtypeStruct((N + C, T), jnp.int32),
            jax.ShapeDtypeStruct((1, 16), jnp.int32),
        ],
    )(xt, br, v)


def _sc_hist_body(a_hbm, cnt_hbm, h_hbm, abufa, abufb, hist, hist2, cbuf,
                  sema, semb):
    cid = lax.axis_index("c")
    sid = lax.axis_index("s")
    wid = sid * 2 + cid
    tg = wid // NBG
    bg = wid % NBG

    pltpu.sync_copy(cnt_hbm.at[0], cbuf)
    iota16 = lax.broadcasted_iota(jnp.int32, (16,), 0)
    cb = cbuf[...]
    zero = jnp.int32(0)
    b1 = jnp.sum(jnp.where(iota16 == 0, cb, zero))
    b2 = jnp.sum(jnp.where(iota16 == 1, cb, zero))
    b3 = jnp.sum(jnp.where(iota16 == 2, cb, zero))
    start = jnp.where(bg == 0, zero, jnp.where(bg == 1, b1, jnp.where(bg == 2, b2, b3)))
    end = jnp.where(bg == 0, b1, jnp.where(bg == 1, b2, jnp.where(bg == 2, b3, jnp.int32(N))))

    ones16 = jnp.ones((16,), jnp.float32)
    trash16 = jnp.full((16,), HROWS * 16, jnp.int32) + iota16

    nch = (end - start + (C - 1)) // C
    npair = jnp.maximum(jnp.int32(1), (nch + 1) // 2)

    def issue(k, buf, sem):
        p0 = jnp.minimum(start + k * C, jnp.int32(N))
        pltpu.make_async_copy(
            a_hbm.at[pl.ds(p0, C), pl.ds(tg * 16, 16)], buf, sem).start()

    def process(k, buf, sem):
        pltpu.make_async_copy(
            a_hbm.at[pl.ds(0, C), pl.ds(0, 16)], buf, sem).wait()
        cnt = jnp.clip(end - (start + k * C), 0, C)

        def fill(j, carry2):
            buf[j] = trash16
            return carry2

        lax.fori_loop(cnt, C, fill, 0)

        @pl.loop(0, C, step=16)
        def _(j):
            avs = [buf[j + u] for u in range(16)]
            for a in avs:
                plsc.addupdate_scatter(hist, [a], ones16)

    issue(jnp.int32(0), abufa, sema)
    issue(jnp.int32(1), abufb, semb)

    zeros16 = jnp.zeros((16,), jnp.float32)

    @pl.loop(0, HROWS * 16 + 16, step=16)
    def _(i):
        hist[pl.ds(i, 16)] = zeros16

    def pair_body(m, carry):
        more = m + 1 < npair
        process(2 * m, abufa, sema)

        @pl.when(more)
        def _():
            issue(2 * m + 2, abufa, sema)

        process(2 * m + 1, abufb, semb)

        @pl.when(more)
        def _():
            issue(2 * m + 3, abufb, semb)

        return carry

    lax.fori_loop(0, npair, pair_body, 0)

    for bl in range(16):
        def csum(r, acc):
            acc = acc + hist[pl.ds((bl * R + r) * 16, 16)]
            hist2[bl * R + r] = acc
            return acc

        lax.fori_loop(0, R, csum, jnp.zeros((16,), jnp.float32))

    pltpu.sync_copy(hist2, h_hbm.at[pl.ds(bg * HROWS, HROWS), tg, :])


@functools.cache
def _sc_hist():
    mesh = plsc.VectorSubcoreMesh(core_axis_name="c", subcore_axis_name="s")
    return pl.kernel(
        _sc_hist_body,
        out_type=jax.ShapeDtypeStruct((NBG * HROWS, NTG, 16), jnp.float32),
        mesh=mesh,
        compiler_params=pltpu.CompilerParams(
            use_tc_tiling_on_sc=False, needs_layout_passes=False),
        scratch_types=[
            pltpu.VMEM((C, 16), jnp.int32),
            pltpu.VMEM((C, 16), jnp.int32),
            pltpu.VMEM((HROWS * 16 + 16,), jnp.float32),
            pltpu.VMEM((HROWS, 16), jnp.float32),
            pltpu.VMEM((16,), jnp.int32),
            pltpu.SemaphoreType.DMA,
            pltpu.SemaphoreType.DMA,
        ],
    )


def kernel(x, batch, v):
    a, counts = _tc1(x.T, batch[None, :], v)
    h = _sc_hist()(a, counts)
    return h.reshape(B, R, T)


# speedup vs baseline: 225.8011x; 1.0317x over previous
import functools

import jax
import jax.numpy as jnp
from jax import lax
from jax.experimental import pallas as pl
from jax.experimental.pallas import tpu as pltpu
from jax.experimental.pallas import tpu_sc as plsc

N = 262144
D = 3
T = 128
R = 128
RAD = 1.1
B = 64

P = 16384
C = 1536
NTG = 8
NBG = 4
HROWS = 16 * R
BBLK = 8


def _tc1_body(xt_ref, b_ref, v_ref, a_ref, cnt_ref):
    i = pl.program_id(0)
    xt = xt_ref[...]
    v = v_ref[...]
    br = b_ref[...]
    k1 = R / (2.0 * RAD)
    dn_t = (((0,), (0,)), ((), ()))
    bloc = (br & (16 - 1)).astype(jnp.float32)
    lhs = jnp.concatenate([xt, jnp.ones((1, P), jnp.float32), bloc],
                          axis=0)
    tlpat = (lax.broadcasted_iota(jnp.int32, (1, T), 1) & 15).astype(jnp.float32)
    vs = jnp.concatenate([v * k1, jnp.full((1, T), R / 2.0),
                          jnp.zeros((1, T), jnp.float32)], axis=0)
    vb = jnp.concatenate([jnp.zeros((3, T), jnp.float32), tlpat,
                          jnp.full((1, T), float(R * 16))], axis=0)
    sb = lax.dot_general(lhs, jnp.concatenate([vs, vb], axis=1), dn_t,
                         preferred_element_type=jnp.float32)
    s = sb[:, :T]
    base = sb[:, T:]
    a_ref[...] = (base + jnp.floor(jnp.clip(s, 0.0, R - 1.0)) * 16.0
                  ).astype(jnp.int32)
    thr = (lax.broadcasted_iota(jnp.int32, (16, 1), 0) + 1) * 16
    bo_t = (br < thr).astype(jnp.float32)
    c = lax.dot_general(jnp.ones((1, P), jnp.float32), bo_t,
                        (((1,), (1,)), ((), ())),
                        preferred_element_type=jnp.float32)

    @pl.when(i == 0)
    def _():
        cnt_ref[...] = jnp.zeros_like(cnt_ref)

    cnt_ref[...] += c.astype(jnp.int32)


def _tc1(xt, br, v):
    return pl.pallas_call(
        _tc1_body,
        grid=(N // P,),
        in_specs=[
            pl.BlockSpec((D, P), lambda i: (0, i)),
            pl.BlockSpec((1, P), lambda i: (0, i)),
            pl.BlockSpec((D, T), lambda i: (0, 0)),
        ],
        out_specs=[
            pl.BlockSpec((P, T), lambda i: (i, 0)),
            pl.BlockSpec((1, 16), lambda i: (0, 0)),
        ],
        out_shape=[
            jax.ShapeDtypeStruct((N + C, T), jnp.int32),
            jax.ShapeDtypeStruct((1, 16), jnp.int32),
        ],
    )(xt, br, v)


def _sc_hist_body(a_hbm, cnt_hbm, h_hbm, abufa, abufb, hist, hist2, cbuf,
                  sema, semb):
    cid = lax.axis_index("c")
    sid = lax.axis_index("s")
    wid = sid * 2 + cid
    tg = wid // NBG
    bg = wid % NBG

    pltpu.sync_copy(cnt_hbm.at[0], cbuf)
    iota16 = lax.broadcasted_iota(jnp.int32, (16,), 0)
    cb = cbuf[...]
    zero = jnp.int32(0)
    b1 = jnp.sum(jnp.where(iota16 == 0, cb, zero))
    b2 = jnp.sum(jnp.where(iota16 == 1, cb, zero))
    b3 = jnp.sum(jnp.where(iota16 == 2, cb, zero))
    start = jnp.where(bg == 0, zero, jnp.where(bg == 1, b1, jnp.where(bg == 2, b2, b3)))
    end = jnp.where(bg == 0, b1, jnp.where(bg == 1, b2, jnp.where(bg == 2, b3, jnp.int32(N))))

    ones16 = jnp.ones((16,), jnp.float32)
    trash16 = jnp.full((16,), HROWS * 16, jnp.int32) + iota16

    nch = (end - start + (C - 1)) // C
    npair = jnp.maximum(jnp.int32(1), (nch + 1) // 2)

    def issue(k, buf, sem):
        p0 = jnp.minimum(start + k * C, jnp.int32(N))
        pltpu.make_async_copy(
            a_hbm.at[pl.ds(p0, C), pl.ds(tg * 16, 16)], buf, sem).start()

    def process(k, buf, sem):
        pltpu.make_async_copy(
            a_hbm.at[pl.ds(0, C), pl.ds(0, 16)], buf, sem).wait()
        cnt = jnp.clip(end - (start + k * C), 0, C)

        def fill(j, carry2):
            buf[j] = trash16
            return carry2

        lax.fori_loop(cnt, C, fill, 0)

        @pl.loop(0, C, step=16)
        def _(j):
            avs = [buf[j + u] for u in range(16)]
            for a in avs:
                plsc.addupdate_scatter(hist, [a], ones16)

    issue(jnp.int32(0), abufa, sema)
    issue(jnp.int32(1), abufb, semb)

    zeros16 = jnp.zeros((16,), jnp.float32)

    @pl.loop(0, HROWS * 16 + 16, step=16)
    def _(i):
        hist[pl.ds(i, 16)] = zeros16

    def pair_body(m, carry):
        more = m + 1 < npair
        process(2 * m, abufa, sema)

        @pl.when(more)
        def _():
            issue(2 * m + 2, abufa, sema)

        process(2 * m + 1, abufb, semb)

        @pl.when(more)
        def _():
            issue(2 * m + 3, abufb, semb)

        return carry

    lax.fori_loop(0, npair, pair_body, 0)

    for bl in range(16):
        def csum(r, acc):
            acc = acc + hist[pl.ds((bl * R + r) * 16, 16)]
            hist2[bl * R + r] = acc
            return acc

        lax.fori_loop(0, R, csum, jnp.zeros((16,), jnp.float32))

    pltpu.sync_copy(hist2, h_hbm.at[pl.ds(bg * HROWS, HROWS), tg, :])


@functools.cache
def _sc_hist():
    mesh = plsc.VectorSubcoreMesh(core_axis_name="c", subcore_axis_name="s")
    return pl.kernel(
        _sc_hist_body,
        out_type=jax.ShapeDtypeStruct((NBG * HROWS, NTG, 16), jnp.float32),
        mesh=mesh,
        compiler_params=pltpu.CompilerParams(
            use_tc_tiling_on_sc=False, needs_layout_passes=False),
        scratch_types=[
            pltpu.VMEM((C, 16), jnp.int32),
            pltpu.VMEM((C, 16), jnp.int32),
            pltpu.VMEM((HROWS * 16 + 16,), jnp.float32),
            pltpu.VMEM((HROWS, 16), jnp.float32),
            pltpu.VMEM((16,), jnp.int32),
            pltpu.SemaphoreType.DMA,
            pltpu.SemaphoreType.DMA,
        ],
    )


def kernel(x, batch, v):
    a, counts = _tc1(x.T, batch[None, :], v)
    h = _sc_hist()(a, counts)
    return h.reshape(B, R, T)


# speedup vs baseline: 225.9610x; 1.0007x over previous
import functools

import jax
import jax.numpy as jnp
from jax import lax
from jax.experimental import pallas as pl
from jax.experimental.pallas import tpu as pltpu
from jax.experimental.pallas import tpu_sc as plsc

N = 262144
D = 3
T = 128
R = 128
RAD = 1.1
B = 64

P = 32768
C = 1536
NTG = 8
NBG = 4
HROWS = 16 * R
BBLK = 8


def _tc1_body(xt_ref, b_ref, v_ref, a_ref, cnt_ref):
    i = pl.program_id(0)
    xt = xt_ref[...]
    v = v_ref[...]
    br = b_ref[...]
    k1 = R / (2.0 * RAD)
    dn_t = (((0,), (0,)), ((), ()))
    bloc = (br & (16 - 1)).astype(jnp.float32)
    lhs = jnp.concatenate([xt, jnp.ones((1, P), jnp.float32), bloc],
                          axis=0)
    tlpat = (lax.broadcasted_iota(jnp.int32, (1, T), 1) & 15).astype(jnp.float32)
    vs = jnp.concatenate([v * k1, jnp.full((1, T), R / 2.0),
                          jnp.zeros((1, T), jnp.float32)], axis=0)
    vb = jnp.concatenate([jnp.zeros((3, T), jnp.float32), tlpat,
                          jnp.full((1, T), float(R * 16))], axis=0)
    sb = lax.dot_general(lhs, jnp.concatenate([vs, vb], axis=1), dn_t,
                         preferred_element_type=jnp.float32)
    s = sb[:, :T]
    base = sb[:, T:]
    a_ref[...] = (base + jnp.floor(jnp.clip(s, 0.0, R - 1.0)) * 16.0
                  ).astype(jnp.int32)
    thr = (lax.broadcasted_iota(jnp.int32, (16, 1), 0) + 1) * 16
    bo_t = (br < thr).astype(jnp.float32)
    c = lax.dot_general(jnp.ones((1, P), jnp.float32), bo_t,
                        (((1,), (1,)), ((), ())),
                        preferred_element_type=jnp.float32)

    @pl.when(i == 0)
    def _():
        cnt_ref[...] = jnp.zeros_like(cnt_ref)

    cnt_ref[...] += c.astype(jnp.int32)


def _tc1(xt, br, v):
    return pl.pallas_call(
        _tc1_body,
        grid=(N // P,),
        in_specs=[
            pl.BlockSpec((D, P), lambda i: (0, i)),
            pl.BlockSpec((1, P), lambda i: (0, i)),
            pl.BlockSpec((D, T), lambda i: (0, 0)),
        ],
        out_specs=[
            pl.BlockSpec((P, T), lambda i: (i, 0)),
            pl.BlockSpec((1, 16), lambda i: (0, 0)),
        ],
        out_shape=[
            jax.ShapeDtypeStruct((N + C, T), jnp.int32),
            jax.ShapeDtypeStruct((1, 16), jnp.int32),
        ],
    )(xt, br, v)


def _sc_hist_body(a_hbm, cnt_hbm, h_hbm, abufa, abufb, hist, hist2, cbuf,
                  sema, semb):
    cid = lax.axis_index("c")
    sid = lax.axis_index("s")
    wid = sid * 2 + cid
    tg = wid // NBG
    bg = wid % NBG

    pltpu.sync_copy(cnt_hbm.at[0], cbuf)
    iota16 = lax.broadcasted_iota(jnp.int32, (16,), 0)
    cb = cbuf[...]
    zero = jnp.int32(0)
    b1 = jnp.sum(jnp.where(iota16 == 0, cb, zero))
    b2 = jnp.sum(jnp.where(iota16 == 1, cb, zero))
    b3 = jnp.sum(jnp.where(iota16 == 2, cb, zero))
    start = jnp.where(bg == 0, zero, jnp.where(bg == 1, b1, jnp.where(bg == 2, b2, b3)))
    end = jnp.where(bg == 0, b1, jnp.where(bg == 1, b2, jnp.where(bg == 2, b3, jnp.int32(N))))

    ones16 = jnp.ones((16,), jnp.float32)
    trash16 = jnp.full((16,), HROWS * 16, jnp.int32) + iota16

    nch = (end - start + (C - 1)) // C
    npair = jnp.maximum(jnp.int32(1), (nch + 1) // 2)

    def issue(k, buf, sem):
        p0 = jnp.minimum(start + k * C, jnp.int32(N))
        pltpu.make_async_copy(
            a_hbm.at[pl.ds(p0, C), pl.ds(tg * 16, 16)], buf, sem).start()

    def process(k, buf, sem):
        pltpu.make_async_copy(
            a_hbm.at[pl.ds(0, C), pl.ds(0, 16)], buf, sem).wait()
        cnt = jnp.clip(end - (start + k * C), 0, C)

        def fill(j, carry2):
            buf[j] = trash16
            return carry2

        lax.fori_loop(cnt, C, fill, 0)

        @pl.loop(0, C, step=16)
        def _(j):
            avs = [buf[j + u] for u in range(16)]
            for a in avs:
                plsc.addupdate_scatter(hist, [a], ones16)

    issue(jnp.int32(0), abufa, sema)
    issue(jnp.int32(1), abufb, semb)

    zeros16 = jnp.zeros((16,), jnp.float32)

    @pl.loop(0, HROWS * 16 + 16, step=16)
    def _(i):
        hist[pl.ds(i, 16)] = zeros16

    def pair_body(m, carry):
        more = m + 1 < npair
        process(2 * m, abufa, sema)

        @pl.when(more)
        def _():
            issue(2 * m + 2, abufa, sema)

        process(2 * m + 1, abufb, semb)

        @pl.when(more)
        def _():
            issue(2 * m + 3, abufb, semb)

        return carry

    lax.fori_loop(0, npair, pair_body, 0)

    for bl in range(16):
        def csum(r, acc):
            acc = acc + hist[pl.ds((bl * R + r) * 16, 16)]
            hist2[bl * R + r] = acc
            return acc

        lax.fori_loop(0, R, csum, jnp.zeros((16,), jnp.float32))

    pltpu.sync_copy(hist2, h_hbm.at[pl.ds(bg * HROWS, HROWS), tg, :])


@functools.cache
def _sc_hist():
    mesh = plsc.VectorSubcoreMesh(core_axis_name="c", subcore_axis_name="s")
    return pl.kernel(
        _sc_hist_body,
        out_type=jax.ShapeDtypeStruct((NBG * HROWS, NTG, 16), jnp.float32),
        mesh=mesh,
        compiler_params=pltpu.CompilerParams(
            use_tc_tiling_on_sc=False, needs_layout_passes=False),
        scratch_types=[
            pltpu.VMEM((C, 16), jnp.int32),
            pltpu.VMEM((C, 16), jnp.int32),
            pltpu.VMEM((HROWS * 16 + 16,), jnp.float32),
            pltpu.VMEM((HROWS, 16), jnp.float32),
            pltpu.VMEM((16,), jnp.int32),
            pltpu.SemaphoreType.DMA,
            pltpu.SemaphoreType.DMA,
        ],
    )


def kernel(x, batch, v):
    a, counts = _tc1(x.T, batch[None, :], v)
    h = _sc_hist()(a, counts)
    return h.reshape(B, R, T)
